# baseline (device time: 20592 ns/iter reference)
import jax
import jax.numpy as jnp
from jax import lax
from jax.experimental import pallas as pl
from jax.experimental.pallas import tpu as pltpu

T = 512
TG = T // 4
C = TG // 2
V_SHARD = 4096
D = 512


def kernel(ids, E):
    def body(ids_ref, e_ref, out_ref, evmem, zbuf, sbuf,
             esem, osem, zsend, zrecv, bsend, brecv):
        my_x = lax.axis_index("x")
        my_y = lax.axis_index("y")
        my_z = lax.axis_index("z")
        my_g = my_x * 2 + my_y
        z_peer = (my_x, my_y, 1 - my_z)
        xy_peer = {
            1: (my_x, 1 - my_y, my_z),
            2: (1 - my_x, my_y, my_z),
            3: (1 - my_x, 1 - my_y, my_z),
        }

        estream = pltpu.make_async_copy(e_ref, evmem, esem)
        estream.start()

        barrier = pltpu.get_barrier_semaphore()
        for dev in [z_peer] + list(xy_peer.values()):
            pl.semaphore_signal(
                barrier, inc=1, device_id=dev,
                device_id_type=pl.DeviceIdType.MESH,
            )
        pl.semaphore_wait(barrier, 4)
        estream.wait()

        cols = lax.broadcasted_iota(jnp.int32, (C, V_SHARD), 1)
        zx = []
        partials = []
        for c in range(2):
            my_ids = ids_ref[pl.ds(my_g * TG + c * C, C), :]
            onehot = (cols == (my_ids - my_z * V_SHARD)).astype(jnp.bfloat16)
            partial = jnp.dot(onehot, evmem[:, :],
                              preferred_element_type=jnp.float32)
            partials.append(partial)
            zbuf[0, c, :, :] = partial.astype(jnp.bfloat16)
            r = pltpu.make_async_remote_copy(
                src_ref=zbuf.at[0, c], dst_ref=zbuf.at[1, c],
                send_sem=zsend.at[c], recv_sem=zrecv.at[c],
                device_id=z_peer, device_id_type=pl.DeviceIdType.MESH,
            )
            r.start()
            zx.append(r)

        bx = []
        for c in range(2):
            zx[c].wait()
            reduced = (
                partials[c] + zbuf[1, c, :, :].astype(jnp.float32)
            ).astype(jnp.bfloat16)
            sbuf[c, :, :] = reduced
            pltpu.make_async_copy(
                sbuf.at[c], out_ref.at[pl.ds(my_g * TG + c * C, C), :], osem
            ).start()
            for k in (1, 2, 3):
                r = pltpu.make_async_remote_copy(
                    src_ref=sbuf.at[c],
                    dst_ref=out_ref.at[pl.ds(my_g * TG + c * C, C), :],
                    send_sem=bsend.at[c, k], recv_sem=brecv.at[c, k],
                    device_id=xy_peer[k],
                    device_id_type=pl.DeviceIdType.MESH,
                )
                r.start()
                bx.append(r)

        for c in range(2):
            pltpu.make_async_copy(
                sbuf.at[c], out_ref.at[pl.ds(my_g * TG + c * C, C), :], osem
            ).wait()
        for r in bx:
            r.wait()

    out = pl.pallas_call(
        body,
        out_shape=jax.ShapeDtypeStruct((T, D), jnp.bfloat16),
        in_specs=[
            pl.BlockSpec(memory_space=pltpu.MemorySpace.VMEM),
            pl.BlockSpec(memory_space=pl.ANY),
        ],
        out_specs=pl.BlockSpec(memory_space=pl.ANY),
        scratch_shapes=[
            pltpu.VMEM((V_SHARD, D), jnp.bfloat16),
            pltpu.VMEM((2, 2, C, D), jnp.bfloat16),
            pltpu.VMEM((2, C, D), jnp.bfloat16),
            pltpu.SemaphoreType.DMA,
            pltpu.SemaphoreType.DMA,
            pltpu.SemaphoreType.DMA((2,)),
            pltpu.SemaphoreType.DMA((2,)),
            pltpu.SemaphoreType.DMA((2, 4)),
            pltpu.SemaphoreType.DMA((2, 4)),
        ],
        compiler_params=pltpu.CompilerParams(collective_id=0),
    )(
        ids.reshape(T, 1),
        pltpu.with_memory_space_constraint(
            E.astype(jnp.bfloat16), pltpu.MemorySpace.HBM
        ),
    )
    return out


# device time: 17135 ns/iter; 1.2018x vs baseline; 1.2018x over previous
import jax
import jax.numpy as jnp
from jax import lax
from jax.experimental import pallas as pl
from jax.experimental.pallas import tpu as pltpu

T = 512
TG = T // 4
NC = 4
C = TG // NC
V_SHARD = 4096
D = 512


def kernel(ids, E):
    def body(ids_ref, e_ref, out_ref, evmem, zbuf, sbuf,
             esem, osem, zsend, zrecv, bsend, brecv):
        my_x = lax.axis_index("x")
        my_y = lax.axis_index("y")
        my_z = lax.axis_index("z")
        my_g = my_x * 2 + my_y
        z_peer = (my_x, my_y, 1 - my_z)
        xy_peer = {
            1: (my_x, 1 - my_y, my_z),
            2: (1 - my_x, my_y, my_z),
            3: (1 - my_x, 1 - my_y, my_z),
        }

        estream = pltpu.make_async_copy(e_ref, evmem, esem)
        estream.start()

        barrier = pltpu.get_barrier_semaphore()
        for dev in [z_peer] + list(xy_peer.values()):
            pl.semaphore_signal(
                barrier, inc=1, device_id=dev,
                device_id_type=pl.DeviceIdType.MESH,
            )
        pl.semaphore_wait(barrier, 4)
        estream.wait()

        cols = lax.broadcasted_iota(jnp.int32, (C, V_SHARD), 1)
        zx = []
        partials = []
        for c in range(NC):
            my_ids = ids_ref[pl.ds(my_g * TG + c * C, C), :]
            onehot = (cols == (my_ids - my_z * V_SHARD)).astype(jnp.float32)
            partial = jnp.dot(onehot, evmem[:, :],
                              preferred_element_type=jnp.float32)
            partials.append(partial)
            zbuf[0, c, :, :] = partial.astype(jnp.bfloat16)
            r = pltpu.make_async_remote_copy(
                src_ref=zbuf.at[0, c], dst_ref=zbuf.at[1, c],
                send_sem=zsend.at[c], recv_sem=zrecv.at[c],
                device_id=z_peer, device_id_type=pl.DeviceIdType.MESH,
            )
            r.start()
            zx.append(r)

        bx = []
        for c in range(NC):
            zx[c].wait()
            reduced = (
                partials[c] + zbuf[1, c, :, :].astype(jnp.float32)
            ).astype(jnp.bfloat16)
            sbuf[c, :, :] = reduced
            pltpu.make_async_copy(
                sbuf.at[c], out_ref.at[pl.ds(my_g * TG + c * C, C), :], osem
            ).start()
            for k in (1, 2, 3):
                r = pltpu.make_async_remote_copy(
                    src_ref=sbuf.at[c],
                    dst_ref=out_ref.at[pl.ds(my_g * TG + c * C, C), :],
                    send_sem=bsend.at[c, k], recv_sem=brecv.at[c, k],
                    device_id=xy_peer[k],
                    device_id_type=pl.DeviceIdType.MESH,
                )
                r.start()
                bx.append(r)

        for c in range(NC):
            pltpu.make_async_copy(
                sbuf.at[c], out_ref.at[pl.ds(my_g * TG + c * C, C), :], osem
            ).wait()
        for r in bx:
            r.wait()

    out = pl.pallas_call(
        body,
        out_shape=jax.ShapeDtypeStruct((T, D), jnp.bfloat16),
        in_specs=[
            pl.BlockSpec(memory_space=pltpu.MemorySpace.VMEM),
            pl.BlockSpec(memory_space=pl.ANY),
        ],
        out_specs=pl.BlockSpec(memory_space=pl.ANY),
        scratch_shapes=[
            pltpu.VMEM((V_SHARD, D), jnp.float32),
            pltpu.VMEM((2, NC, C, D), jnp.bfloat16),
            pltpu.VMEM((NC, C, D), jnp.bfloat16),
            pltpu.SemaphoreType.DMA,
            pltpu.SemaphoreType.DMA,
            pltpu.SemaphoreType.DMA((NC,)),
            pltpu.SemaphoreType.DMA((NC,)),
            pltpu.SemaphoreType.DMA((NC, 4)),
            pltpu.SemaphoreType.DMA((NC, 4)),
        ],
        compiler_params=pltpu.CompilerParams(collective_id=0),
    )(
        ids.reshape(T, 1),
        pltpu.with_memory_space_constraint(E, pltpu.MemorySpace.HBM),
    )
    return pltpu.with_memory_space_constraint(out, pltpu.MemorySpace.HBM)


# device time: 15244 ns/iter; 1.3508x vs baseline; 1.1240x over previous
import jax
import jax.numpy as jnp
from jax import lax
from jax.experimental import pallas as pl
from jax.experimental.pallas import tpu as pltpu

T = 512
TG = T // 4
NC = 4
C = TG // NC
V_SHARD = 4096
D = 512


def kernel(ids, E):
    def body(ids_sref, e_ref, out_ref, gbuf, zbuf, sbuf,
             gsem, osem, zsend, zrecv, bsend, brecv):
        my_x = lax.axis_index("x")
        my_y = lax.axis_index("y")
        my_z = lax.axis_index("z")
        my_g = my_x * 2 + my_y
        z_peer = (my_x, my_y, 1 - my_z)
        xy_peer = {
            1: (my_x, 1 - my_y, my_z),
            2: (1 - my_x, my_y, my_z),
            3: (1 - my_x, 1 - my_y, my_z),
        }

        barrier = pltpu.get_barrier_semaphore()
        for dev in [z_peer] + list(xy_peer.values()):
            pl.semaphore_signal(
                barrier, inc=1, device_id=dev,
                device_id_type=pl.DeviceIdType.MESH,
            )

        gbuf[:, :] = jnp.zeros((TG, D), jnp.float32)

        def row(i):
            raw = ids_sref[my_g * TG + i] - my_z * V_SHARD
            return raw, jnp.logical_and(raw >= 0, raw < V_SHARD)

        for i in range(TG):
            raw, valid = row(i)

            @pl.when(valid)
            def _():
                pltpu.make_async_copy(
                    e_ref.at[pl.ds(raw, 1), :], gbuf.at[pl.ds(i, 1), :], gsem
                ).start()

        pl.semaphore_wait(barrier, 4)

        for i in range(TG):
            raw, valid = row(i)

            @pl.when(valid)
            def _():
                pltpu.make_async_copy(
                    e_ref.at[pl.ds(0, 1), :], gbuf.at[pl.ds(i, 1), :], gsem
                ).wait()

        zx = []
        for c in range(NC):
            zbuf[0, c, :, :] = gbuf[pl.ds(c * C, C), :].astype(jnp.bfloat16)
            r = pltpu.make_async_remote_copy(
                src_ref=zbuf.at[0, c], dst_ref=zbuf.at[1, c],
                send_sem=zsend.at[c], recv_sem=zrecv.at[c],
                device_id=z_peer, device_id_type=pl.DeviceIdType.MESH,
            )
            r.start()
            zx.append(r)

        bx = []
        for c in range(NC):
            zx[c].wait()
            reduced = (
                gbuf[pl.ds(c * C, C), :]
                + zbuf[1, c, :, :].astype(jnp.float32)
            ).astype(jnp.bfloat16)
            sbuf[c, :, :] = reduced
            pltpu.make_async_copy(
                sbuf.at[c], out_ref.at[pl.ds(my_g * TG + c * C, C), :], osem
            ).start()
            for k in (1, 2, 3):
                r = pltpu.make_async_remote_copy(
                    src_ref=sbuf.at[c],
                    dst_ref=out_ref.at[pl.ds(my_g * TG + c * C, C), :],
                    send_sem=bsend.at[c, k], recv_sem=brecv.at[c, k],
                    device_id=xy_peer[k],
                    device_id_type=pl.DeviceIdType.MESH,
                )
                r.start()
                bx.append(r)

        for c in range(NC):
            pltpu.make_async_copy(
                sbuf.at[c], out_ref.at[pl.ds(my_g * TG + c * C, C), :], osem
            ).wait()
        for r in bx:
            r.wait()

    out = pl.pallas_call(
        body,
        out_shape=jax.ShapeDtypeStruct((T, D), jnp.bfloat16),
        in_specs=[
            pl.BlockSpec(memory_space=pltpu.MemorySpace.SMEM),
            pl.BlockSpec(memory_space=pl.ANY),
        ],
        out_specs=pl.BlockSpec(memory_space=pl.ANY),
        scratch_shapes=[
            pltpu.VMEM((TG, D), jnp.float32),
            pltpu.VMEM((2, NC, C, D), jnp.bfloat16),
            pltpu.VMEM((NC, C, D), jnp.bfloat16),
            pltpu.SemaphoreType.DMA,
            pltpu.SemaphoreType.DMA,
            pltpu.SemaphoreType.DMA((NC,)),
            pltpu.SemaphoreType.DMA((NC,)),
            pltpu.SemaphoreType.DMA((NC, 4)),
            pltpu.SemaphoreType.DMA((NC, 4)),
        ],
        compiler_params=pltpu.CompilerParams(collective_id=0),
    )(ids, pltpu.with_memory_space_constraint(E, pltpu.MemorySpace.HBM))
    return out


# device time: 14684 ns/iter; 1.4023x vs baseline; 1.0381x over previous
import jax
import jax.numpy as jnp
from jax import lax
from jax.experimental import pallas as pl
from jax.experimental.pallas import tpu as pltpu

T = 512
TG = T // 4
NC = 4
C = TG // NC
V_SHARD = 4096
D = 512


def kernel(ids, E):
    def body(ids_sref, e_ref, out_ref, gbuf, zbuf, sbuf,
             gsem, osem, zsend, zrecv, bsend, brecv):
        my_x = lax.axis_index("x")
        my_y = lax.axis_index("y")
        my_z = lax.axis_index("z")
        my_g = my_x * 2 + my_y
        z_peer = (my_x, my_y, 1 - my_z)
        xy_peer = {
            1: (my_x, 1 - my_y, my_z),
            2: (1 - my_x, my_y, my_z),
            3: (1 - my_x, 1 - my_y, my_z),
        }

        barrier = pltpu.get_barrier_semaphore()
        for dev in [z_peer] + list(xy_peer.values()):
            pl.semaphore_signal(
                barrier, inc=1, device_id=dev,
                device_id_type=pl.DeviceIdType.MESH,
            )

        gbuf[:, :] = jnp.zeros((TG, D), jnp.float32)

        def row(i):
            raw = ids_sref[my_g * TG + i] - my_z * V_SHARD
            return raw, jnp.logical_and(raw >= 0, raw < V_SHARD)

        for i in range(TG):
            raw, valid = row(i)

            @pl.when(valid)
            def _():
                pltpu.make_async_copy(
                    e_ref.at[pl.ds(raw, 1), :], gbuf.at[pl.ds(i, 1), :],
                    gsem.at[i // C]
                ).start()

        pl.semaphore_wait(barrier, 4)

        zx = []
        for c in range(NC):
            for i in range(c * C, (c + 1) * C):
                raw, valid = row(i)

                @pl.when(valid)
                def _():
                    pltpu.make_async_copy(
                        e_ref.at[pl.ds(0, 1), :], gbuf.at[pl.ds(i, 1), :],
                        gsem.at[i // C]
                    ).wait()

            zbuf[0, c, :, :] = gbuf[pl.ds(c * C, C), :].astype(jnp.bfloat16)
            r = pltpu.make_async_remote_copy(
                src_ref=zbuf.at[0, c], dst_ref=zbuf.at[1, c],
                send_sem=zsend.at[c], recv_sem=zrecv.at[c],
                device_id=z_peer, device_id_type=pl.DeviceIdType.MESH,
            )
            r.start()
            zx.append(r)

        bx = []
        for c in range(NC):
            zx[c].wait()
            reduced = (
                gbuf[pl.ds(c * C, C), :]
                + zbuf[1, c, :, :].astype(jnp.float32)
            ).astype(jnp.bfloat16)
            sbuf[c, :, :] = reduced
            pltpu.make_async_copy(
                sbuf.at[c], out_ref.at[pl.ds(my_g * TG + c * C, C), :], osem
            ).start()
            for k in (1, 2, 3):
                r = pltpu.make_async_remote_copy(
                    src_ref=sbuf.at[c],
                    dst_ref=out_ref.at[pl.ds(my_g * TG + c * C, C), :],
                    send_sem=bsend.at[c, k], recv_sem=brecv.at[c, k],
                    device_id=xy_peer[k],
                    device_id_type=pl.DeviceIdType.MESH,
                )
                r.start()
                bx.append(r)

        for c in range(NC):
            pltpu.make_async_copy(
                sbuf.at[c], out_ref.at[pl.ds(my_g * TG + c * C, C), :], osem
            ).wait()
        for r in bx:
            r.wait()

    out = pl.pallas_call(
        body,
        out_shape=jax.ShapeDtypeStruct((T, D), jnp.bfloat16),
        in_specs=[
            pl.BlockSpec(memory_space=pltpu.MemorySpace.SMEM),
            pl.BlockSpec(memory_space=pl.ANY),
        ],
        out_specs=pl.BlockSpec(memory_space=pltpu.MemorySpace.HBM),
        scratch_shapes=[
            pltpu.VMEM((TG, D), jnp.float32),
            pltpu.VMEM((2, NC, C, D), jnp.bfloat16),
            pltpu.VMEM((NC, C, D), jnp.bfloat16),
            pltpu.SemaphoreType.DMA((NC,)),
            pltpu.SemaphoreType.DMA,
            pltpu.SemaphoreType.DMA((NC,)),
            pltpu.SemaphoreType.DMA((NC,)),
            pltpu.SemaphoreType.DMA((NC, 4)),
            pltpu.SemaphoreType.DMA((NC, 4)),
        ],
        compiler_params=pltpu.CompilerParams(collective_id=0),
    )(ids, pltpu.with_memory_space_constraint(E, pltpu.MemorySpace.HBM))
    return out
